# baseline (speedup 1.0000x reference)
import jax
import jax.numpy as jnp
from jax.experimental import pallas as pl
from jax.experimental.pallas import tpu as pltpu

_C = 4


def _ffn_kernel(x_ref, w1_ref, b1_ref, w2_ref, b2_ref, o_ref):
    j = pl.program_id(0)
    h = jnp.dot(x_ref[0], w1_ref[0], preferred_element_type=jnp.float32)
    h = h + b1_ref[0]
    h = 0.5 * h * (1.0 + jax.lax.erf(h * 0.7071067811865476))
    contrib = jnp.dot(h, w2_ref[0], preferred_element_type=jnp.float32)

    @pl.when(j == 0)
    def _():
        o_ref[0] = contrib + b2_ref[0]

    @pl.when(j > 0)
    def _():
        o_ref[0] = o_ref[0] + contrib


def kernel(x, edge_index, W_gat, att_src, att_dst, bias_gat, ln_gamma,
           ln_beta, W1, b1, W2, b2):
    B, N, D = x.shape
    NE, _, F = W1.shape
    fc = F // _C

    out = pl.pallas_call(
        _ffn_kernel,
        grid=(_C,),
        in_specs=[
            pl.BlockSpec((1, N, D), lambda j: (0, 0, 0)),
            pl.BlockSpec((1, D, fc), lambda j: (0, 0, j)),
            pl.BlockSpec((1, 1, fc), lambda j: (0, 0, j)),
            pl.BlockSpec((1, fc, D), lambda j: (0, j, 0)),
            pl.BlockSpec((1, 1, D), lambda j: (0, 0, 0)),
        ],
        out_specs=pl.BlockSpec((1, N, D), lambda j: (0, 0, 0)),
        out_shape=jax.ShapeDtypeStruct((B, N, D), jnp.float32),
    )(x, W1, b1.reshape(NE, 1, F), W2, b2.reshape(NE, 1, D))
    return out


# speedup vs baseline: 1.0303x; 1.0303x over previous
import jax
import jax.numpy as jnp
from jax.experimental import pallas as pl
from jax.experimental.pallas import tpu as pltpu


def _ffn_kernel(x_ref, w1_ref, b1_ref, w2_ref, b2_ref, o_ref):
    h = jnp.dot(x_ref[0], w1_ref[0], preferred_element_type=jnp.float32)
    h = h + b1_ref[0]
    h = 0.5 * h * (1.0 + jax.lax.erf(h * 0.7071067811865476))
    o = jnp.dot(h, w2_ref[0], preferred_element_type=jnp.float32)
    o_ref[0] = o + b2_ref[0]


def kernel(x, edge_index, W_gat, att_src, att_dst, bias_gat, ln_gamma,
           ln_beta, W1, b1, W2, b2):
    B, N, D = x.shape
    NE, _, F = W1.shape

    tn = 128
    grid = (B * N // tn,)
    out = pl.pallas_call(
        _ffn_kernel,
        grid=grid,
        in_specs=[
            pl.BlockSpec((1, tn, D), lambda i: (0, i, 0)),
            pl.BlockSpec((1, D, F), lambda i: (0, 0, 0)),
            pl.BlockSpec((1, 1, F), lambda i: (0, 0, 0)),
            pl.BlockSpec((1, F, D), lambda i: (0, 0, 0)),
            pl.BlockSpec((1, 1, D), lambda i: (0, 0, 0)),
        ],
        out_specs=pl.BlockSpec((1, tn, D), lambda i: (0, i, 0)),
        out_shape=jax.ShapeDtypeStruct((B, N, D), jnp.float32),
        compiler_params=pltpu.CompilerParams(
            dimension_semantics=("parallel",)),
    )(x, W1, b1.reshape(NE, 1, F), W2, b2.reshape(NE, 1, D))
    return out


# speedup vs baseline: 1.1419x; 1.1083x over previous
import jax
import jax.numpy as jnp
from jax.experimental import pallas as pl
from jax.experimental.pallas import tpu as pltpu


def _ffn_kernel(x_ref, w1_ref, b1_ref, w2_ref, b2_ref, o_ref):
    h = jnp.dot(x_ref[0], w1_ref[0], preferred_element_type=jnp.float32)
    h = h + b1_ref[0]
    h = 0.5 * h * (1.0 + jax.lax.erf(h * 0.7071067811865476))
    o = jnp.dot(h, w2_ref[0], preferred_element_type=jnp.float32)
    o_ref[0] = o + b2_ref[0]


def kernel(x, edge_index, W_gat, att_src, att_dst, bias_gat, ln_gamma,
           ln_beta, W1, b1, W2, b2):
    B, N, D = x.shape
    NE, _, F = W1.shape

    tn = 512
    grid = (B * N // tn,)
    out = pl.pallas_call(
        _ffn_kernel,
        grid=grid,
        in_specs=[
            pl.BlockSpec((1, tn, D), lambda i: (0, i, 0)),
            pl.BlockSpec((1, D, F), lambda i: (0, 0, 0)),
            pl.BlockSpec((1, 1, F), lambda i: (0, 0, 0)),
            pl.BlockSpec((1, F, D), lambda i: (0, 0, 0)),
            pl.BlockSpec((1, 1, D), lambda i: (0, 0, 0)),
        ],
        out_specs=pl.BlockSpec((1, tn, D), lambda i: (0, i, 0)),
        out_shape=jax.ShapeDtypeStruct((B, N, D), jnp.float32),
        compiler_params=pltpu.CompilerParams(
            dimension_semantics=("parallel",)),
    )(x, W1, b1.reshape(NE, 1, F), W2, b2.reshape(NE, 1, D))
    return out
